# Initial kernel scaffold; baseline (speedup 1.0000x reference)
import jax
import jax.numpy as jnp
from jax.experimental import pallas as pl


def kernel(encoded_input_data, tra1, tra2, tra3, z, edge_index, edge_weight, W1, g1, b1, W2, g2, b2, W3, g3, b3, W4, g4, b4, W5, g5, b5, cluster):
    raise NotImplementedError("write your pallas kernel here")


# speedup vs baseline: 3.4999x; 3.4999x over previous
import functools

import jax
import jax.numpy as jnp
from jax import lax
from jax.experimental import pallas as pl
from jax.experimental.pallas import tpu as pltpu
from jax.experimental.pallas import tpu_sc as plsc

N = 10000
NPAD = 10240
E = 320000
NW = 32
C = 128
KC = 80
EPAD = NW * C * KC
ROWS_PER_TILE = NPAD // 16
SIGMA = 0.5


def _make_spmm(D):
  mesh = plsc.VectorSubcoreMesh(core_axis_name="c", subcore_axis_name="s",
                                num_cores=2, num_subcores=16)

  @functools.partial(
      pl.kernel,
      out_type=jax.ShapeDtypeStruct((2 * NPAD, D), jnp.float32),
      mesh=mesh,
      scratch_types=[
          pltpu.VMEM((KC, C), jnp.int32),
          pltpu.VMEM((KC, C), jnp.int32),
          pltpu.VMEM((KC, C), jnp.float32),
          pltpu.VMEM((C, D), jnp.float32),
          pltpu.VMEM_SHARED((NPAD, D), jnp.float32),
          pltpu.SemaphoreType.DMA,
      ],
      compiler_params=pltpu.CompilerParams(use_tc_tiling_on_sc=False),
  )
  def spmm(x_hbm, src_hbm, dst_hbm, w_hbm, out_hbm,
           src_v, dst_v, w_v, rows_v, acc_s, gsem):
    c = lax.axis_index("c")
    s = lax.axis_index("s")
    wid = s * 2 + c

    zero16 = jnp.zeros((16,), jnp.float32)

    def zrow(i, carry):
      for cb in range(D // 16):
        rows_v[i, pl.ds(cb * 16, 16)] = zero16
      return carry

    lax.fori_loop(0, C, zrow, 0, unroll=2)

    base = s * ROWS_PER_TILE
    nfull = ROWS_PER_TILE // C
    rem = ROWS_PER_TILE - nfull * C
    for r in range(nfull):
      pltpu.sync_copy(rows_v, acc_s.at[pl.ds(base + r * C, C)])
    if rem:
      pltpu.sync_copy(rows_v.at[pl.ds(0, rem)],
                      acc_s.at[pl.ds(base + nfull * C, rem)])

    ebase = wid * KC
    pltpu.sync_copy(src_hbm.at[pl.ds(ebase, KC)], src_v)
    pltpu.sync_copy(dst_hbm.at[pl.ds(ebase, KC)], dst_v)
    pltpu.sync_copy(w_hbm.at[pl.ds(ebase, KC)], w_v)

    plsc.subcore_barrier()

    def chunk_body(j, carry):
      pltpu.async_copy(x_hbm.at[src_v.at[j]], rows_v, gsem).wait()

      def grp_body(g, cc):
        wvec = w_v[j, pl.ds(g * 16, 16)]
        for l in range(16):
          wl = wvec[l]
          e = g * 16 + l
          for cb in range(D // 16):
            sl = pl.ds(cb * 16, 16)
            rows_v[e, sl] = rows_v[e, sl] * wl
        return cc

      lax.fori_loop(0, C // 16, grp_body, 0)

      pltpu.sync_copy(rows_v, acc_s.at[dst_v.at[j]], add=True)
      return carry

    lax.fori_loop(0, KC, chunk_body, 0)

    plsc.subcore_barrier()

    pltpu.sync_copy(acc_s.at[pl.ds(base, ROWS_PER_TILE)],
                    out_hbm.at[pl.ds(c * NPAD + base, ROWS_PER_TILE)])

  return spmm


_spmm_cache = {}


def _spmm(D):
  if D not in _spmm_cache:
    _spmm_cache[D] = _make_spmm(D)
  return _spmm_cache[D]


def _bn(t, g, b):
  m = jnp.mean(t, axis=0, keepdims=True)
  v = jnp.mean(jnp.square(t - m), axis=0, keepdims=True)
  return g * (t - m) / jnp.sqrt(v + 1e-5) + b


def _tc_mid_body(p_ref, w_ref, g_ref, b_ref, tra_ref, o_ref):
  ssum = p_ref[pl.ds(0, N), :] + p_ref[pl.ds(NPAD, N), :]
  t = jnp.dot(ssum, w_ref[...], preferred_element_type=jnp.float32)
  h = jnp.maximum(_bn(t, g_ref[...], b_ref[...]), 0.0)
  o_ref[...] = (1.0 - SIGMA) * h + SIGMA * tra_ref[...]


def _tc_mid(p, w, g, b, tra):
  return pl.pallas_call(
      _tc_mid_body,
      out_shape=jax.ShapeDtypeStruct((N, 128), jnp.float32),
  )(p, w, g, b, tra)


def _tc_l3_body(p_ref, w_ref, g_ref, b_ref, tra_ref, w4_ref, o_ref):
  ssum = p_ref[pl.ds(0, N), :] + p_ref[pl.ds(NPAD, N), :]
  t = jnp.dot(ssum, w_ref[...], preferred_element_type=jnp.float32)
  h = jnp.maximum(_bn(t, g_ref[...], b_ref[...]), 0.0)
  u = (1.0 - SIGMA) * h + SIGMA * tra_ref[...]
  o_ref[...] = jnp.dot(u, w4_ref[...], preferred_element_type=jnp.float32)


def _tc_l3(p, w, g, b, tra, w4p):
  return pl.pallas_call(
      _tc_l3_body,
      out_shape=jax.ShapeDtypeStruct((N, 16), jnp.float32),
  )(p, w, g, b, tra, w4p)


def _tc_l4_body(p_ref, g_ref, b_ref, z_ref, w5_ref, o_ref):
  ssum = (p_ref[pl.ds(0, N), :] + p_ref[pl.ds(NPAD, N), :])[:, :10]
  h = jnp.maximum(_bn(ssum, g_ref[...], b_ref[...]), 0.0)
  u = (1.0 - SIGMA) * h + SIGMA * z_ref[...]
  o_ref[...] = jnp.dot(u, w5_ref[...], preferred_element_type=jnp.float32)


def _tc_l4(p, g, b, z, w5p):
  return pl.pallas_call(
      _tc_l4_body,
      out_shape=jax.ShapeDtypeStruct((N, 16), jnp.float32),
  )(p, g, b, z, w5p)


def _tc_final_body(p_ref, g_ref, b_ref, z_ref, cl_ref,
                   q_ref, pred_ref, h5_ref):
  ssum = (p_ref[pl.ds(0, N), :] + p_ref[pl.ds(NPAD, N), :])[:, :10]
  h5 = _bn(ssum, g_ref[...], b_ref[...])
  h5_ref[...] = h5
  nrm = jnp.sqrt(jnp.sum(h5 * h5, axis=1, keepdims=True)) + 1e-12
  pred_ref[...] = jax.nn.softmax(h5 / nrm, axis=1)
  z = z_ref[...]
  cl = cl_ref[...]
  z2 = jnp.sum(z * z, axis=1, keepdims=True)
  c2 = jnp.sum(cl * cl, axis=1)
  zc = lax.dot_general(z, cl, (((1,), (1,)), ((), ())),
                       preferred_element_type=jnp.float32)
  d2 = z2 + c2[None, :] - 2.0 * zc
  q = 1.0 / (1.0 + d2)
  q_ref[...] = q / jnp.sum(q, axis=1, keepdims=True)


def _tc_final(p, g, b, z, cl):
  return pl.pallas_call(
      _tc_final_body,
      out_shape=(
          jax.ShapeDtypeStruct((N, 4), jnp.float32),
          jax.ShapeDtypeStruct((N, 10), jnp.float32),
          jax.ShapeDtypeStruct((N, 10), jnp.float32),
      ),
  )(p, g, b, z, cl)


def kernel(encoded_input_data, tra1, tra2, tra3, z, edge_index, edge_weight,
           W1, g1, b1, W2, g2, b2, W3, g3, b3, W4, g4, b4, W5, g5, b5,
           cluster):
  pad = EPAD - E
  src = jnp.concatenate([edge_index[0], jnp.zeros((pad,), jnp.int32)])
  dst = jnp.concatenate([edge_index[1], jnp.zeros((pad,), jnp.int32)])
  wgt = jnp.concatenate([edge_weight, jnp.zeros((pad,), jnp.float32)])
  src2d = src.reshape(NW * KC, C)
  dst2d = dst.reshape(NW * KC, C)
  wgt2d = wgt.reshape(NW * KC, C)

  w4p = jnp.pad(W4, ((0, 0), (0, 6)))
  w5p = jnp.pad(W5, ((0, 0), (0, 6)))

  spmm128 = _spmm(128)
  spmm16 = _spmm(16)
  p1 = spmm128(encoded_input_data, src2d, dst2d, wgt2d)
  u2 = _tc_mid(p1, W1, g1, b1, tra1)
  p2 = spmm128(u2, src2d, dst2d, wgt2d)
  u3 = _tc_mid(p2, W2, g2, b2, tra2)
  p3 = spmm128(u3, src2d, dst2d, wgt2d)
  y4 = _tc_l3(p3, W3, g3, b3, tra3, w4p)
  p4 = spmm16(y4, src2d, dst2d, wgt2d)
  y5 = _tc_l4(p4, g4, b4, z, w5p)
  p5 = spmm16(y5, src2d, dst2d, wgt2d)
  q, pred, h5 = _tc_final(p5, g5, b5, z, cluster)
  return (q, pred, h5)


# speedup vs baseline: 4.6343x; 1.3241x over previous
import functools

import jax
import jax.numpy as jnp
from jax import lax
from jax.experimental import pallas as pl
from jax.experimental.pallas import tpu as pltpu
from jax.experimental.pallas import tpu_sc as plsc

N = 10000
NPAD = 10240
E = 320000
C = 128
NCHUNKS = 2560
EPAD = NCHUNKS * C
ROWS_PER_TILE = NPAD // 16
SIGMA = 0.5
NB = 4


def _make_spmm(dsc, col_split):
  nstage = NCHUNKS // 16 if col_split else NCHUNKS // 32
  mesh = plsc.VectorSubcoreMesh(core_axis_name="c", subcore_axis_name="s",
                                num_cores=2, num_subcores=16)

  @functools.partial(
      pl.kernel,
      out_type=jax.ShapeDtypeStruct((2 * NPAD, dsc), jnp.float32),
      mesh=mesh,
      scratch_types=[
          pltpu.VMEM((nstage, C), jnp.int32),
          pltpu.VMEM((nstage, C), jnp.int32),
          pltpu.VMEM((C, dsc), jnp.float32),
          pltpu.VMEM((C, dsc), jnp.float32),
          pltpu.VMEM((C, dsc), jnp.float32),
          pltpu.VMEM((C, dsc), jnp.float32),
          pltpu.VMEM((C,), jnp.float32),
          pltpu.VMEM((C,), jnp.float32),
          pltpu.VMEM((C,), jnp.float32),
          pltpu.VMEM((C,), jnp.float32),
          pltpu.VMEM_SHARED((NPAD, dsc), jnp.float32),
          [pltpu.SemaphoreType.DMA] * NB,
          [pltpu.SemaphoreType.DMA] * NB,
      ],
      compiler_params=pltpu.CompilerParams(use_tc_tiling_on_sc=False),
  )
  def spmm(x_hbm, src_hbm, dst_hbm, w_hbm, out_hbm,
           src_v, dst_v, r0, r1, r2, r3, w0, w1, w2, w3,
           acc_s, gsems, ssems):
    c = lax.axis_index("c")
    s = lax.axis_index("s")
    bufs = [r0, r1, r2, r3]
    wslots = [w0, w1, w2, w3]

    if col_split:
      ebase = c * NCHUNKS + s * nstage
      wbase = s * nstage
    else:
      ebase = (s * 2 + c) * nstage
      wbase = ebase

    zero16 = jnp.zeros((16,), jnp.float32)

    def zrow(i, carry):
      for cb in range(dsc // 16):
        r0[i, pl.ds(cb * 16, 16)] = zero16
      return carry

    lax.fori_loop(0, C, zrow, 0, unroll=2)

    base = s * ROWS_PER_TILE
    for r in range(ROWS_PER_TILE // C):
      pltpu.sync_copy(r0, acc_s.at[pl.ds(base + r * C, C)])

    pltpu.sync_copy(src_hbm.at[pl.ds(ebase, nstage)], src_v)
    pltpu.sync_copy(dst_hbm.at[pl.ds(wbase, nstage)], dst_v)

    plsc.subcore_barrier()

    def gather(j, b):
      pltpu.async_copy(x_hbm.at[src_v.at[j]], bufs[b], gsems[b])
      pltpu.async_copy(w_hbm.at[pl.ds((wbase + j) * C, C)], wslots[b],
                       gsems[b])

    def gwait(b):
      pltpu.make_async_copy(x_hbm.at[pl.ds(0, C)], bufs[b], gsems[b]).wait()
      pltpu.make_async_copy(w_hbm.at[pl.ds(0, C)], wslots[b], gsems[b]).wait()

    def scatter(j, b):
      pltpu.async_copy(bufs[b], acc_s.at[dst_v.at[j]], ssems[b], add=True)

    def swait(b):
      pltpu.make_async_copy(x_hbm.at[pl.ds(0, C)], bufs[b], ssems[b]).wait()

    def mult(j, b):
      buf = bufs[b]
      wsl = wslots[b]

      def grp_body(g, cc):
        wvec = wsl[pl.ds(g * 16, 16)]
        for l in range(16):
          wl = wvec[l]
          e = g * 16 + l
          for cb in range(dsc // 16):
            sl = pl.ds(cb * 16, 16)
            buf[e, sl] = buf[e, sl] * wl
        return cc

      lax.fori_loop(0, C // 16, grp_body, 0)

    gather(0, 0)
    gather(1, 1)
    for b in range(NB):
      gwait(b)
      mult(b, b)
      scatter(b, b)
      b2 = (b + 2) % NB
      if b >= NB - 2:
        swait(b2)
      gather(b + 2, b2)

    def block_body(g, carry):
      for b in range(NB):
        j = g * NB + b
        gwait(b)
        mult(j, b)
        scatter(j, b)
        b2 = (b + 2) % NB
        swait(b2)
        gather(j + 2, b2)
      return carry

    lax.fori_loop(1, nstage // NB - 1, block_body, 0)

    for b in range(NB):
      j = nstage - NB + b
      gwait(b)
      mult(j, b)
      scatter(j, b)
      if b < NB - 2:
        b2 = (b + 2) % NB
        swait(b2)
        gather(j + 2, b2)
    for b in range(NB):
      swait(b)

    plsc.subcore_barrier()

    pltpu.sync_copy(acc_s.at[pl.ds(base, ROWS_PER_TILE)],
                    out_hbm.at[pl.ds(c * NPAD + base, ROWS_PER_TILE)])

  return spmm


_spmm_cache = {}


def _spmm(dsc, col_split):
  key = (dsc, col_split)
  if key not in _spmm_cache:
    _spmm_cache[key] = _make_spmm(dsc, col_split)
  return _spmm_cache[key]


def _bn(t, g, b):
  m = jnp.mean(t, axis=0, keepdims=True)
  v = jnp.mean(jnp.square(t - m), axis=0, keepdims=True)
  return g * (t - m) / jnp.sqrt(v + 1e-5) + b


def _assemble(p_ref):
  return jnp.concatenate(
      [p_ref[pl.ds(0, N), :], p_ref[pl.ds(NPAD, N), :]], axis=1)


def _split_out(o_ref, u):
  o_ref[pl.ds(0, N), :] = u[:, :64]
  o_ref[pl.ds(N, N), :] = u[:, 64:]


def _tc_mid_body(p_ref, w_ref, g_ref, b_ref, tra_ref, o_ref):
  t = jnp.dot(_assemble(p_ref), w_ref[...], preferred_element_type=jnp.float32)
  h = jnp.maximum(_bn(t, g_ref[...], b_ref[...]), 0.0)
  _split_out(o_ref, (1.0 - SIGMA) * h + SIGMA * tra_ref[...])


def _tc_mid(p, w, g, b, tra):
  return pl.pallas_call(
      _tc_mid_body,
      out_shape=jax.ShapeDtypeStruct((2 * N, 64), jnp.float32),
  )(p, w, g, b, tra)


def _tc_l3_body(p_ref, w_ref, g_ref, b_ref, tra_ref, w4_ref, o_ref):
  t = jnp.dot(_assemble(p_ref), w_ref[...], preferred_element_type=jnp.float32)
  h = jnp.maximum(_bn(t, g_ref[...], b_ref[...]), 0.0)
  u = (1.0 - SIGMA) * h + SIGMA * tra_ref[...]
  o_ref[...] = jnp.dot(u, w4_ref[...], preferred_element_type=jnp.float32)


def _tc_l3(p, w, g, b, tra, w4p):
  return pl.pallas_call(
      _tc_l3_body,
      out_shape=jax.ShapeDtypeStruct((N, 16), jnp.float32),
  )(p, w, g, b, tra, w4p)


def _tc_l4_body(p_ref, g_ref, b_ref, z_ref, w5_ref, o_ref):
  ssum = (p_ref[pl.ds(0, N), :] + p_ref[pl.ds(NPAD, N), :])[:, :10]
  h = jnp.maximum(_bn(ssum, g_ref[...], b_ref[...]), 0.0)
  u = (1.0 - SIGMA) * h + SIGMA * z_ref[...]
  o_ref[...] = jnp.dot(u, w5_ref[...], preferred_element_type=jnp.float32)


def _tc_l4(p, g, b, z, w5p):
  return pl.pallas_call(
      _tc_l4_body,
      out_shape=jax.ShapeDtypeStruct((N, 16), jnp.float32),
  )(p, g, b, z, w5p)


def _tc_final_body(p_ref, g_ref, b_ref, z_ref, cl_ref,
                   q_ref, pred_ref, h5_ref):
  ssum = (p_ref[pl.ds(0, N), :] + p_ref[pl.ds(NPAD, N), :])[:, :10]
  h5 = _bn(ssum, g_ref[...], b_ref[...])
  h5_ref[...] = h5
  nrm = jnp.sqrt(jnp.sum(h5 * h5, axis=1, keepdims=True)) + 1e-12
  pred_ref[...] = jax.nn.softmax(h5 / nrm, axis=1)
  z = z_ref[...]
  cl = cl_ref[...]
  z2 = jnp.sum(z * z, axis=1, keepdims=True)
  c2 = jnp.sum(cl * cl, axis=1)
  zc = lax.dot_general(z, cl, (((1,), (1,)), ((), ())),
                       preferred_element_type=jnp.float32)
  d2 = z2 + c2[None, :] - 2.0 * zc
  q = 1.0 / (1.0 + d2)
  q_ref[...] = q / jnp.sum(q, axis=1, keepdims=True)


def _tc_final(p, g, b, z, cl):
  return pl.pallas_call(
      _tc_final_body,
      out_shape=(
          jax.ShapeDtypeStruct((N, 4), jnp.float32),
          jax.ShapeDtypeStruct((N, 10), jnp.float32),
          jax.ShapeDtypeStruct((N, 10), jnp.float32),
      ),
  )(p, g, b, z, cl)


def kernel(encoded_input_data, tra1, tra2, tra3, z, edge_index, edge_weight,
           W1, g1, b1, W2, g2, b2, W3, g3, b3, W4, g4, b4, W5, g5, b5,
           cluster):
  pad = EPAD - E
  src = jnp.concatenate([edge_index[0], jnp.zeros((pad,), jnp.int32)])
  dst = jnp.concatenate([edge_index[1], jnp.zeros((pad,), jnp.int32)])
  wgt = jnp.concatenate([edge_weight, jnp.zeros((pad,), jnp.float32)])
  src2d = src.reshape(NCHUNKS, C)
  dst2d = dst.reshape(NCHUNKS, C)
  src_cs = jnp.concatenate([src2d, src2d + N], axis=0)
  wflat = wgt

  w4p = jnp.pad(W4, ((0, 0), (0, 6)))
  w5p = jnp.pad(W5, ((0, 0), (0, 6)))

  spmm128 = _spmm(64, True)
  spmm16 = _spmm(16, False)

  x0 = jnp.concatenate([encoded_input_data[:, :64],
                        encoded_input_data[:, 64:]], axis=0)
  p1 = spmm128(x0, src_cs, dst2d, wflat)
  u2 = _tc_mid(p1, W1, g1, b1, tra1)
  p2 = spmm128(u2, src_cs, dst2d, wflat)
  u3 = _tc_mid(p2, W2, g2, b2, tra2)
  p3 = spmm128(u3, src_cs, dst2d, wflat)
  y4 = _tc_l3(p3, W3, g3, b3, tra3, w4p)
  p4 = spmm16(y4, src2d, dst2d, wflat)
  y5 = _tc_l4(p4, g4, b4, z, w5p)
  p5 = spmm16(y5, src2d, dst2d, wflat)
  q, pred, h5 = _tc_final(p5, g5, b5, z, cluster)
  return (q, pred, h5)
